# baseline (speedup 1.0000x reference)
import functools

import jax
import jax.numpy as jnp
from jax.experimental import pallas as pl
from jax.experimental.pallas import tpu as pltpu

N = 10000
E = 10000
NUM_GRAPHS = 8

ROW_BLK = 200


def _kcl_kernel(c_ref, v_ref, inj_ref, out_ref):
    i = pl.program_id(0)
    kcl = jnp.dot(c_ref[...], v_ref[...],
                  preferred_element_type=jnp.float32) - inj_ref[...]

    @pl.when(i == 0)
    def _():
        out_ref[...] = jnp.zeros_like(out_ref)

    out_ref[...] += jnp.sum(kcl * kcl).reshape(1, 1)


def _kcl_sq_sum(C, v, inj):
    out = pl.pallas_call(
        _kcl_kernel,
        grid=(N // ROW_BLK,),
        in_specs=[
            pl.BlockSpec((ROW_BLK, N), lambda i: (i, 0)),
            pl.BlockSpec((N, 1), lambda i: (0, 0)),
            pl.BlockSpec((ROW_BLK, 1), lambda i: (i, 0)),
        ],
        out_specs=pl.BlockSpec((1, 1), lambda i: (0, 0)),
        out_shape=jax.ShapeDtypeStruct((1, 1), jnp.float32),
    )(C, v.reshape(N, 1), inj.reshape(N, 1))
    return out[0, 0]


def _gcn(x, row, col, W, b, n):
    h = x @ W
    deg = jax.ops.segment_sum(jnp.ones_like(col, dtype=h.dtype), col, num_segments=n)
    dinv = jnp.where(deg > 0, deg ** -0.5, 0.0)
    norm = dinv[row] * dinv[col]
    msg = h[row] * norm[:, None]
    return jax.ops.segment_sum(msg, col, num_segments=n) + b


def _mlp(h, W1, b1, W2, b2):
    return (jax.nn.relu(h @ W1 + b1) @ W2 + b2)[:, 0]


def kernel(x, edge_index, edge_attr, conductance_matrix, net_injection, line_currents, batch, W1, b1, W2, b2, W3, b3, Ws1, bs1, Ws2, bs2, Wv1, bv1, Wv2, bv2, Wf1, bf1, Wf2, bf2):
    n = x.shape[0]
    row0, col0 = edge_index[0], edge_index[1]
    loop = jnp.arange(n)
    row = jnp.concatenate([row0, loop])
    col = jnp.concatenate([col0, loop])
    h = x.astype(jnp.float32)
    for (W, b) in ((W1, b1), (W2, b2), (W3, b3)):
        h = jax.nn.relu(_gcn(h, row, col, W, b, n))
    ones = jnp.ones((n,), h.dtype)
    cnt = jax.ops.segment_sum(ones, batch, num_segments=NUM_GRAPHS)
    sums = jax.ops.segment_sum(h, batch, num_segments=NUM_GRAPHS)
    graph_emb = sums / jnp.maximum(cnt, 1.0)[:, None]
    switch_scores = _mlp(graph_emb, Ws1, bs1, Ws2, bs2)
    decisions = jax.nn.sigmoid(switch_scores)
    qubo_loss = jnp.sum(decisions * decisions)
    voltages = _mlp(h, Wv1, bv1, Wv2, bv2)
    kirchhoff_kcl = _kcl_sq_sum(conductance_matrix, voltages, net_injection) / n
    R = edge_attr[:, 0]
    vdrop = voltages[row0] - voltages[col0]
    kvl = vdrop - R * line_currents
    kirchhoff_loss = kirchhoff_kcl + jnp.mean(kvl ** 2)
    radial_loss = (decisions.sum() - (n - 1)) ** 2 / n
    flows = _mlp(h, Wf1, bf1, Wf2, bf2)
    lf = vdrop - edge_attr[:, 0] * flows
    loadflow_loss = jnp.mean(lf ** 2)
    total_physics_loss = kirchhoff_loss + radial_loss + loadflow_loss
    return (decisions, qubo_loss, total_physics_loss)


# speedup vs baseline: 2.7259x; 2.7259x over previous
import functools

import jax
import jax.numpy as jnp
from jax import lax
from jax.experimental import pallas as pl
from jax.experimental.pallas import tpu as pltpu
from jax.experimental.pallas import tpu_sc as plsc

N = 10000
E = 10000
NUM_GRAPHS = 8

NC = 2
NS = 16
NW = NC * NS
EPAD = 10240
EPT = EPAD // NW
CHUNK = 80
NCHUNK = EPT // CHUNK
NPAD = 10240
RPT = NPAD // NS
ZR = 64

ROW_BLK = 200
TC_BLK = 1000


def _mesh():
    return plsc.VectorSubcoreMesh(core_axis_name="c", subcore_axis_name="s")


_SC_PARAMS = pltpu.CompilerParams(use_tc_tiling_on_sc=False,
                                  needs_layout_passes=False)


def _zero_vmem(buf, rows, width):
    z = jnp.zeros((16,), jnp.float32)
    for r in range(rows):
        for q in range(width // 16):
            buf[r, pl.ds(q * 16, 16)] = z


def _deg_body(cidx_hbm, out_hbm, cidx_v, ones_v, zeros_v, acc_sh, sem):
    c = lax.axis_index("c")
    s = lax.axis_index("s")
    wid = s * NC + c
    pltpu.sync_copy(cidx_hbm.at[wid], cidx_v)
    _zero_vmem(zeros_v, ZR, 16)
    e1 = jnp.where(lax.iota(jnp.int32, 16) == 0, 1.0, 0.0)
    for r in range(CHUNK):
        ones_v[r, pl.ds(0, 16)] = e1
    for z in range(RPT // ZR):
        pltpu.sync_copy(zeros_v, acc_sh.at[pl.ds(s * RPT + z * ZR, ZR)])
    plsc.subcore_barrier()
    for j in range(NCHUNK):
        pltpu.sync_copy(ones_v, acc_sh.at[cidx_v.at[j]], add=True)
    plsc.subcore_barrier()
    pltpu.sync_copy(acc_sh.at[pl.ds(s * RPT, RPT)],
                    out_hbm.at[c, pl.ds(s * RPT, RPT)])


def _sc_degree(cidx):
    return pl.kernel(
        _deg_body,
        out_type=jax.ShapeDtypeStruct((NC, NPAD, 16), jnp.float32),
        mesh=_mesh(),
        scratch_types=[
            pltpu.VMEM((NCHUNK, CHUNK), jnp.int32),
            pltpu.VMEM((CHUNK, 16), jnp.float32),
            pltpu.VMEM((ZR, 16), jnp.float32),
            pltpu.VMEM_SHARED((NPAD, 16), jnp.float32),
            pltpu.SemaphoreType.DMA,
        ],
        compiler_params=_SC_PARAMS,
    )(cidx)


def _scat_body(F, g_hbm, ridx_hbm, cidx_hbm, out_hbm,
               ridx_v, cidx_v, rows_v, zeros_v, acc_sh, sem):
    c = lax.axis_index("c")
    s = lax.axis_index("s")
    wid = s * NC + c
    pltpu.sync_copy(ridx_hbm.at[wid], ridx_v)
    pltpu.sync_copy(cidx_hbm.at[wid], cidx_v)
    copies = []
    for j in range(NCHUNK):
        copies.append(pltpu.async_copy(
            g_hbm.at[ridx_v.at[j]], rows_v.at[pl.ds(j * CHUNK, CHUNK)], sem))
    _zero_vmem(zeros_v, ZR, F)
    for z in range(RPT // ZR):
        pltpu.sync_copy(zeros_v, acc_sh.at[pl.ds(s * RPT + z * ZR, ZR)])
    for cp in copies:
        cp.wait()
    plsc.subcore_barrier()
    for j in range(NCHUNK):
        pltpu.sync_copy(rows_v.at[pl.ds(j * CHUNK, CHUNK)],
                        acc_sh.at[cidx_v.at[j]], add=True)
    plsc.subcore_barrier()
    pltpu.sync_copy(acc_sh.at[pl.ds(s * RPT, RPT)],
                    out_hbm.at[c, pl.ds(s * RPT, RPT)])


def _sc_scatter(g, ridx, cidx, F):
    return pl.kernel(
        functools.partial(_scat_body, F),
        out_type=jax.ShapeDtypeStruct((NC, NPAD, F), jnp.float32),
        mesh=_mesh(),
        scratch_types=[
            pltpu.VMEM((NCHUNK, CHUNK), jnp.int32),
            pltpu.VMEM((NCHUNK, CHUNK), jnp.int32),
            pltpu.VMEM((EPT, F), jnp.float32),
            pltpu.VMEM((ZR, F), jnp.float32),
            pltpu.VMEM_SHARED((NPAD, F), jnp.float32),
            pltpu.SemaphoreType.DMA,
        ],
        compiler_params=_SC_PARAMS,
    )(g, ridx, cidx)


def _eloss_body(volt_hbm, ridx_hbm, cidx_hbm, rlin_hbm, ilin_hbm, flin_hbm,
                out_hbm, volt_v, ridx_v, cidx_v, r_v, i_v, f_v, res_v, sem):
    c = lax.axis_index("c")
    s = lax.axis_index("s")
    wid = s * NC + c
    base = wid * EPT
    pltpu.sync_copy(volt_hbm, volt_v)
    pltpu.sync_copy(ridx_hbm.at[pl.ds(base, EPT)], ridx_v)
    pltpu.sync_copy(cidx_hbm.at[pl.ds(base, EPT)], cidx_v)
    pltpu.sync_copy(rlin_hbm.at[pl.ds(base, EPT)], r_v)
    pltpu.sync_copy(ilin_hbm.at[pl.ds(base, EPT)], i_v)
    pltpu.sync_copy(flin_hbm.at[pl.ds(base, EPT)], f_v)
    kvl_acc = jnp.zeros((16,), jnp.float32)
    lf_acc = jnp.zeros((16,), jnp.float32)
    for k in range(EPT // 16):
        ri = ridx_v[pl.ds(k * 16, 16)]
        ci = cidx_v[pl.ds(k * 16, 16)]
        vr = plsc.load_gather(volt_v, [ri])
        vc = plsc.load_gather(volt_v, [ci])
        vd = vr - vc
        rr = r_v[pl.ds(k * 16, 16)]
        kvl = vd - rr * i_v[pl.ds(k * 16, 16)]
        lf = vd - rr * f_v[pl.ds(k * 16, 16)]
        kvl_acc = kvl_acc + kvl * kvl
        lf_acc = lf_acc + lf * lf
    res_v[0, pl.ds(0, 16)] = kvl_acc
    res_v[1, pl.ds(0, 16)] = lf_acc
    pltpu.sync_copy(res_v, out_hbm.at[wid])


def _sc_edge_loss(volt, ridx_flat, cidx_flat, r_lin, i_lin, f_lin):
    return pl.kernel(
        _eloss_body,
        out_type=jax.ShapeDtypeStruct((NW, 2, 16), jnp.float32),
        mesh=_mesh(),
        scratch_types=[
            pltpu.VMEM((N,), jnp.float32),
            pltpu.VMEM((EPT,), jnp.int32),
            pltpu.VMEM((EPT,), jnp.int32),
            pltpu.VMEM((EPT,), jnp.float32),
            pltpu.VMEM((EPT,), jnp.float32),
            pltpu.VMEM((EPT,), jnp.float32),
            pltpu.VMEM((2, 16), jnp.float32),
            pltpu.SemaphoreType.DMA,
        ],
        compiler_params=_SC_PARAMS,
    )(volt, ridx_flat, cidx_flat, r_lin, i_lin, f_lin)


def _pre_body(x_ref, w_ref, d0_ref, d1_ref, g_ref, dinv_ref):
    deg = 1.0 + d0_ref[0, :, 0:1] + d1_ref[0, :, 0:1]
    dinv = lax.rsqrt(deg)
    dinv_ref[...] = dinv
    g_ref[...] = jnp.dot(x_ref[...], w_ref[...],
                         preferred_element_type=jnp.float32) * dinv


def _tc_pre(x, W1, deg):
    grid = (N // TC_BLK,)
    return pl.pallas_call(
        _pre_body,
        grid=grid,
        in_specs=[
            pl.BlockSpec((TC_BLK, 128), lambda i: (i, 0)),
            pl.BlockSpec((128, 64), lambda i: (0, 0)),
            pl.BlockSpec((1, TC_BLK, 16), lambda i: (0, i, 0)),
            pl.BlockSpec((1, TC_BLK, 16), lambda i: (1, i, 0)),
        ],
        out_specs=[
            pl.BlockSpec((TC_BLK, 64), lambda i: (i, 0)),
            pl.BlockSpec((TC_BLK, 1), lambda i: (i, 0)),
        ],
        out_shape=[
            jax.ShapeDtypeStruct((N, 64), jnp.float32),
            jax.ShapeDtypeStruct((N, 1), jnp.float32),
        ],
    )(x, W1, deg, deg)


def _layer_body(s0_ref, s1_ref, g_ref, dinv_ref, b_ref, w_ref, out_ref):
    dinv = dinv_ref[...]
    h = jax.nn.relu(dinv * (g_ref[...] + s0_ref[0] + s1_ref[0]) + b_ref[...])
    out_ref[...] = jnp.dot(h, w_ref[...],
                           preferred_element_type=jnp.float32) * dinv


def _tc_layer(scat, g, dinv, b, Wn, F, Fn):
    grid = (N // TC_BLK,)
    return pl.pallas_call(
        _layer_body,
        grid=grid,
        in_specs=[
            pl.BlockSpec((1, TC_BLK, F), lambda i: (0, i, 0)),
            pl.BlockSpec((1, TC_BLK, F), lambda i: (1, i, 0)),
            pl.BlockSpec((TC_BLK, F), lambda i: (i, 0)),
            pl.BlockSpec((TC_BLK, 1), lambda i: (i, 0)),
            pl.BlockSpec((1, F), lambda i: (0, 0)),
            pl.BlockSpec((F, Fn), lambda i: (0, 0)),
        ],
        out_specs=pl.BlockSpec((TC_BLK, Fn), lambda i: (i, 0)),
        out_shape=jax.ShapeDtypeStruct((N, Fn), jnp.float32),
    )(scat, scat, g, dinv, b.reshape(1, F), Wn)


def _heads_body(s0_ref, s1_ref, g_ref, dinv_ref, b3_ref, batch_ref,
                wv1_ref, bv1_ref, wv2_ref, bv2_ref,
                wf1_ref, bf1_ref, wf2_ref, bf2_ref,
                ws1_ref, bs1_ref, ws2_ref, bs2_ref,
                volt_ref, flow_ref, dec_ref, qubo_ref, radial_ref,
                pool_acc, cnt_acc):
    i = pl.program_id(0)
    h3 = jax.nn.relu(dinv_ref[...] * (g_ref[...] + s0_ref[0] + s1_ref[0])
                     + b3_ref[...])
    hv = jax.nn.relu(jnp.dot(h3, wv1_ref[...],
                             preferred_element_type=jnp.float32) + bv1_ref[...])
    volt_ref[...] = jnp.dot(hv, wv2_ref[...],
                            preferred_element_type=jnp.float32) + bv2_ref[...]
    hf = jax.nn.relu(jnp.dot(h3, wf1_ref[...],
                             preferred_element_type=jnp.float32) + bf1_ref[...])
    flow_ref[...] = jnp.dot(hf, wf2_ref[...],
                            preferred_element_type=jnp.float32) + bf2_ref[...]

    iota8 = lax.broadcasted_iota(jnp.int32, (1, NUM_GRAPHS), 1)
    onehot = (batch_ref[...] == iota8).astype(jnp.float32)
    dims = (((0,), (0,)), ((), ()))
    pool = lax.dot_general(onehot, h3, dims,
                           preferred_element_type=jnp.float32)
    ones_col = jnp.ones((TC_BLK, 1), jnp.float32)
    cnt = lax.dot_general(onehot, ones_col, dims,
                          preferred_element_type=jnp.float32)

    @pl.when(i == 0)
    def _():
        pool_acc[...] = jnp.zeros_like(pool_acc)
        cnt_acc[...] = jnp.zeros_like(cnt_acc)

    pool_acc[...] += pool
    cnt_acc[...] += cnt

    @pl.when(i == pl.num_programs(0) - 1)
    def _():
        emb = pool_acc[...] / jnp.maximum(cnt_acc[...], 1.0)
        hs = jax.nn.relu(jnp.dot(emb, ws1_ref[...],
                                 preferred_element_type=jnp.float32)
                         + bs1_ref[...])
        scores = jnp.dot(hs, ws2_ref[...],
                         preferred_element_type=jnp.float32) + bs2_ref[...]
        dec = jax.nn.sigmoid(scores)
        dec_ref[...] = dec
        qubo_ref[...] = jnp.sum(dec * dec).reshape(1, 1)
        dsum = jnp.sum(dec)
        radial_ref[...] = ((dsum - (N - 1)) ** 2 / N).reshape(1, 1)


def _tc_heads(scat, g3, dinv, b3, batch2d,
              Wv1, bv1, Wv2, bv2, Wf1, bf1, Wf2, bf2, Ws1, bs1, Ws2, bs2):
    grid = (N // TC_BLK,)
    cst = lambda i: (0, 0)
    return pl.pallas_call(
        _heads_body,
        grid=grid,
        in_specs=[
            pl.BlockSpec((1, TC_BLK, 16), lambda i: (0, i, 0)),
            pl.BlockSpec((1, TC_BLK, 16), lambda i: (1, i, 0)),
            pl.BlockSpec((TC_BLK, 16), lambda i: (i, 0)),
            pl.BlockSpec((TC_BLK, 1), lambda i: (i, 0)),
            pl.BlockSpec((1, 16), cst),
            pl.BlockSpec((TC_BLK, 1), lambda i: (i, 0)),
            pl.BlockSpec((16, 64), cst),
            pl.BlockSpec((1, 64), cst),
            pl.BlockSpec((64, 1), cst),
            pl.BlockSpec((1, 1), cst),
            pl.BlockSpec((16, 64), cst),
            pl.BlockSpec((1, 64), cst),
            pl.BlockSpec((64, 1), cst),
            pl.BlockSpec((1, 1), cst),
            pl.BlockSpec((16, 64), cst),
            pl.BlockSpec((1, 64), cst),
            pl.BlockSpec((64, 1), cst),
            pl.BlockSpec((1, 1), cst),
        ],
        out_specs=[
            pl.BlockSpec((TC_BLK, 1), lambda i: (i, 0)),
            pl.BlockSpec((TC_BLK, 1), lambda i: (i, 0)),
            pl.BlockSpec((NUM_GRAPHS, 1), cst),
            pl.BlockSpec((1, 1), cst),
            pl.BlockSpec((1, 1), cst),
        ],
        out_shape=[
            jax.ShapeDtypeStruct((N, 1), jnp.float32),
            jax.ShapeDtypeStruct((N, 1), jnp.float32),
            jax.ShapeDtypeStruct((NUM_GRAPHS, 1), jnp.float32),
            jax.ShapeDtypeStruct((1, 1), jnp.float32),
            jax.ShapeDtypeStruct((1, 1), jnp.float32),
        ],
        scratch_shapes=[
            pltpu.VMEM((NUM_GRAPHS, 16), jnp.float32),
            pltpu.VMEM((NUM_GRAPHS, 1), jnp.float32),
        ],
    )(scat, scat, g3, dinv, b3.reshape(1, 16), batch2d,
      Wv1, bv1.reshape(1, 64), Wv2, bv2.reshape(1, 1),
      Wf1, bf1.reshape(1, 64), Wf2, bf2.reshape(1, 1),
      Ws1, bs1.reshape(1, 64), Ws2, bs2.reshape(1, 1))


def _kcl_kernel(c_ref, v_ref, inj_ref, out_ref):
    i = pl.program_id(0)
    kcl = jnp.dot(c_ref[...], v_ref[...],
                  preferred_element_type=jnp.float32) - inj_ref[...]

    @pl.when(i == 0)
    def _():
        out_ref[...] = jnp.zeros_like(out_ref)

    out_ref[...] += jnp.sum(kcl * kcl).reshape(1, 1)


def _kcl_sq_sum(C, v, inj):
    out = pl.pallas_call(
        _kcl_kernel,
        grid=(N // ROW_BLK,),
        in_specs=[
            pl.BlockSpec((ROW_BLK, N), lambda i: (i, 0)),
            pl.BlockSpec((N, 1), lambda i: (0, 0)),
            pl.BlockSpec((ROW_BLK, 1), lambda i: (i, 0)),
        ],
        out_specs=pl.BlockSpec((1, 1), lambda i: (0, 0)),
        out_shape=jax.ShapeDtypeStruct((1, 1), jnp.float32),
    )(C, v, inj.reshape(N, 1))
    return out[0, 0]


def kernel(x, edge_index, edge_attr, conductance_matrix, net_injection, line_currents, batch, W1, b1, W2, b2, W3, b3, Ws1, bs1, Ws2, bs2, Wv1, bv1, Wv2, bv2, Wf1, bf1, Wf2, bf2):
    row0, col0 = edge_index[0], edge_index[1]
    pad = EPAD - E
    ridx_flat = jnp.concatenate([row0, jnp.zeros((pad,), jnp.int32)])
    cidx_flat = jnp.concatenate([col0, jnp.zeros((pad,), jnp.int32)])
    cidx_pad = jnp.concatenate([col0, jnp.full((pad,), N, jnp.int32)])
    ridx = ridx_flat.reshape(NW, NCHUNK, CHUNK)
    cidx = cidx_pad.reshape(NW, NCHUNK, CHUNK)

    deg = _sc_degree(cidx)

    g1, dinv = _tc_pre(x, W1, deg)
    s1 = _sc_scatter(g1, ridx, cidx, 64)
    g2 = _tc_layer(s1, g1, dinv, b1, W2, 64, 32)
    s2 = _sc_scatter(g2, ridx, cidx, 32)
    g3 = _tc_layer(s2, g2, dinv, b2, W3, 32, 16)
    s3 = _sc_scatter(g3, ridx, cidx, 16)

    volt, flow, dec, qubo, radial = _tc_heads(
        s3, g3, dinv, b3, batch.reshape(N, 1),
        Wv1, bv1, Wv2, bv2, Wf1, bf1, Wf2, bf2, Ws1, bs1, Ws2, bs2)

    zpad = jnp.zeros((pad,), jnp.float32)
    r_lin = jnp.concatenate([edge_attr[:, 0], zpad])
    i_lin = jnp.concatenate([line_currents, zpad])
    f_lin = jnp.concatenate([flow[:, 0], zpad])

    eloss = _sc_edge_loss(volt[:, 0], ridx_flat, cidx_flat,
                          r_lin, i_lin, f_lin)
    kcl_sq = _kcl_sq_sum(conductance_matrix, volt, net_injection)

    kvl_sum = jnp.sum(eloss[:, 0, :])
    lf_sum = jnp.sum(eloss[:, 1, :])
    total_physics_loss = (kcl_sq / N + kvl_sum / E + lf_sum / E
                          + radial[0, 0])
    decisions = dec[:, 0]
    qubo_loss = qubo[0, 0]
    return (decisions, qubo_loss, total_physics_loss)
